# baseline (speedup 1.0000x reference)
import functools

import jax
import jax.numpy as jnp
import numpy as np
from jax.experimental import pallas as pl
from jax.experimental.pallas import tpu as pltpu
from jax.experimental.pallas import tpu_sc as plsc

B, H, E, F = 32, 2048, 8, 4096
BF = 512
NF = F // BF
BHO = 512
NHO = H // BHO

_MASK_VAL = float(np.finfo(np.float32).min)


def _logits_kernel(x_ref, gw_ref, out_ref):
    out_ref[...] = jax.lax.dot_general(
        gw_ref[...], x_ref[...], (((0,), (1,)), ((), ())),
        preferred_element_type=jnp.float32)


@jax.jit
def _gate_logits(x, gate_w):
    return pl.pallas_call(
        _logits_kernel,
        out_shape=jax.ShapeDtypeStruct((E, B), jnp.float32),
    )(x, gate_w)


def _sc_top2_kernel(logits_hbm, out_hbm, l_v, w_v):
    wid = jax.lax.axis_index("s") * 2 + jax.lax.axis_index("c")

    @pl.when(wid == 0)
    def _():
        pltpu.sync_copy(logits_hbm, l_v)
        mask16 = jnp.full((16,), _MASK_VAL, jnp.float32)
        one16 = jnp.full((16,), 1.0, jnp.float32)
        zero16 = jnp.zeros((16,), jnp.float32)
        for h in range(B // 16):
            le = [l_v[pl.ds(e * B + h * 16, 16)] for e in range(E)]
            m0 = le[0]
            for e in range(1, E):
                m0 = jnp.maximum(m0, le[e])
            masked = [jnp.where(le[e] == m0, mask16, le[e]) for e in range(E)]
            m1 = masked[0]
            for e in range(1, E):
                m1 = jnp.maximum(m1, masked[e])
            pre = one16 / (one16 + jnp.exp(m1 - m0))
            for e in range(E):
                c0 = jnp.where(le[e] == m0, one16, zero16)
                c1 = jnp.where(le[e] == m1, one16, zero16)
                w_v[pl.ds(e * B + h * 16, 16)] = c0 * pre - c1 * (pre - one16)
        pltpu.sync_copy(w_v, out_hbm)


@jax.jit
def _sc_top2(logits_flat):
    return pl.kernel(
        _sc_top2_kernel,
        mesh=plsc.VectorSubcoreMesh(core_axis_name="c", subcore_axis_name="s"),
        out_type=jax.ShapeDtypeStruct((E * B,), jnp.float32),
        scratch_types=[
            pltpu.VMEM((E * B,), jnp.float32),
            pltpu.VMEM((E * B,), jnp.float32),
        ],
    )(logits_flat)


def _experts_kernel(x_ref, w1_ref, w3_ref, w2_ref, out_ref, xb_ref):
    e = pl.program_id(0)
    j = pl.program_id(1)

    @pl.when((e == 0) & (j == 0))
    def _precast():
        xb_ref[...] = x_ref[...].astype(jnp.bfloat16)

    @pl.when(j == 0)
    def _zero():
        out_ref[...] = jnp.zeros_like(out_ref)

    xb = xb_ref[...]
    h1 = jnp.dot(xb, w1_ref[0].astype(jnp.bfloat16),
                 preferred_element_type=jnp.float32)
    h3 = jnp.dot(xb, w3_ref[0].astype(jnp.bfloat16),
                 preferred_element_type=jnp.float32)
    hidden = (h1 * jax.nn.sigmoid(h1)) * h3
    out_ref[0] += jnp.dot(hidden.astype(jnp.bfloat16),
                          w2_ref[0].astype(jnp.bfloat16),
                          preferred_element_type=jnp.float32)


@jax.jit
def _experts(x, w1, w3, w2):
    return pl.pallas_call(
        _experts_kernel,
        grid=(E, NF),
        in_specs=[
            pl.BlockSpec((B, H), lambda e, j: (0, 0)),
            pl.BlockSpec((1, H, BF), lambda e, j: (e, 0, j)),
            pl.BlockSpec((1, H, BF), lambda e, j: (e, 0, j)),
            pl.BlockSpec((1, BF, H), lambda e, j: (e, j, 0)),
        ],
        out_specs=pl.BlockSpec((1, B, H), lambda e, j: (e, 0, 0)),
        out_shape=jax.ShapeDtypeStruct((E, B, H), jnp.float32),
        scratch_shapes=[pltpu.VMEM((B, H), jnp.bfloat16)],
        compiler_params=pltpu.CompilerParams(
            dimension_semantics=("arbitrary", "arbitrary"),
        ),
    )(x, w1, w3, w2)


def _combine_kernel(eout_ref, w_ref, out_ref):
    out_ref[...] = jnp.sum(eout_ref[...] * w_ref[...][..., None], axis=0)


@jax.jit
def _combine(eout, wall_t):
    return pl.pallas_call(
        _combine_kernel,
        grid=(NHO,),
        in_specs=[
            pl.BlockSpec((E, B, BHO), lambda j: (0, 0, j)),
            pl.BlockSpec((E, B), lambda j: (0, 0)),
        ],
        out_specs=pl.BlockSpec((B, BHO), lambda j: (0, j)),
        out_shape=jax.ShapeDtypeStruct((B, H), jnp.float32),
    )(eout, wall_t)


def kernel(inputs, gate_w, w1, w3, w2):
    x = inputs.reshape(B, H)
    logits_t = _gate_logits(x, gate_w)
    wall_t = _sc_top2(logits_t.reshape(E * B)).reshape(E, B)
    eout = _experts(x, w1, w3, w2)
    out = _combine(eout, wall_t)
    return out.reshape(1, 1, B, H)


# speedup vs baseline: 1.0106x; 1.0106x over previous
import functools

import jax
import jax.numpy as jnp
import numpy as np
from jax.experimental import pallas as pl
from jax.experimental.pallas import tpu as pltpu
from jax.experimental.pallas import tpu_sc as plsc

B, H, E, F = 32, 2048, 8, 4096
BF = 512
NF = F // BF
BHO = 512
NHO = H // BHO

_MASK_VAL = float(np.finfo(np.float32).min)


def _logits_kernel(x_ref, gw_ref, out_ref):
    out_ref[...] = jax.lax.dot_general(
        gw_ref[...], x_ref[...], (((0,), (1,)), ((), ())),
        preferred_element_type=jnp.float32)


@jax.jit
def _gate_logits(x, gate_w):
    return pl.pallas_call(
        _logits_kernel,
        out_shape=jax.ShapeDtypeStruct((E, B), jnp.float32),
    )(x, gate_w)


def _sc_top2_kernel(logits_hbm, out_hbm, l_v, w_v):
    wid = jax.lax.axis_index("s") * 2 + jax.lax.axis_index("c")

    @pl.when(wid == 0)
    def _():
        pltpu.sync_copy(logits_hbm, l_v)
        mask16 = jnp.full((16,), _MASK_VAL, jnp.float32)
        one16 = jnp.full((16,), 1.0, jnp.float32)
        zero16 = jnp.zeros((16,), jnp.float32)
        for h in range(B // 16):
            le = [l_v[pl.ds(e * B + h * 16, 16)] for e in range(E)]
            m0 = le[0]
            for e in range(1, E):
                m0 = jnp.maximum(m0, le[e])
            masked = [jnp.where(le[e] == m0, mask16, le[e]) for e in range(E)]
            m1 = masked[0]
            for e in range(1, E):
                m1 = jnp.maximum(m1, masked[e])
            pre = one16 / (one16 + jnp.exp(m1 - m0))
            for e in range(E):
                c0 = jnp.where(le[e] == m0, one16, zero16)
                c1 = jnp.where(le[e] == m1, one16, zero16)
                w_v[pl.ds(e * B + h * 16, 16)] = c0 * pre - c1 * (pre - one16)
        pltpu.sync_copy(w_v, out_hbm)


@jax.jit
def _sc_top2(logits_flat):
    return pl.kernel(
        _sc_top2_kernel,
        mesh=plsc.VectorSubcoreMesh(core_axis_name="c", subcore_axis_name="s"),
        out_type=jax.ShapeDtypeStruct((E * B,), jnp.float32),
        scratch_types=[
            pltpu.VMEM((E * B,), jnp.float32),
            pltpu.VMEM((E * B,), jnp.float32),
        ],
    )(logits_flat)


def _experts_kernel(x_ref, wall_ref, w1_ref, w3_ref, w2_ref, out_ref,
                    scale_ref, xb_ref):
    e = pl.program_id(0)
    j = pl.program_id(1)

    @pl.when((e == 0) & (j == 0))
    def _precast():
        xb_ref[...] = x_ref[...].astype(jnp.bfloat16)
        out_ref[...] = jnp.zeros_like(out_ref)

    @pl.when(j == 0)
    def _scale():
        onehot = jax.lax.broadcasted_iota(jnp.int32, (1, E), 1) == e
        scale_ref[...] = jnp.sum(jnp.where(onehot, wall_ref[...], 0.0),
                                 axis=1, keepdims=True)

    xb = xb_ref[...]
    h1 = jnp.dot(xb, w1_ref[0].astype(jnp.bfloat16),
                 preferred_element_type=jnp.float32)
    h3 = jnp.dot(xb, w3_ref[0].astype(jnp.bfloat16),
                 preferred_element_type=jnp.float32)
    hidden = (h1 * jax.nn.sigmoid(h1)) * h3
    hidden = hidden * scale_ref[...]
    out_ref[...] += jnp.dot(hidden.astype(jnp.bfloat16),
                            w2_ref[0].astype(jnp.bfloat16),
                            preferred_element_type=jnp.float32)


@jax.jit
def _experts(x, wall, w1, w3, w2):
    return pl.pallas_call(
        _experts_kernel,
        grid=(E, NF),
        in_specs=[
            pl.BlockSpec((B, H), lambda e, j: (0, 0)),
            pl.BlockSpec((B, E), lambda e, j: (0, 0)),
            pl.BlockSpec((1, H, BF), lambda e, j: (e, 0, j)),
            pl.BlockSpec((1, H, BF), lambda e, j: (e, 0, j)),
            pl.BlockSpec((1, BF, H), lambda e, j: (e, j, 0)),
        ],
        out_specs=pl.BlockSpec((B, H), lambda e, j: (0, 0)),
        out_shape=jax.ShapeDtypeStruct((B, H), jnp.float32),
        scratch_shapes=[
            pltpu.VMEM((B, 1), jnp.float32),
            pltpu.VMEM((B, H), jnp.bfloat16),
        ],
        compiler_params=pltpu.CompilerParams(
            dimension_semantics=("arbitrary", "arbitrary"),
        ),
    )(x, wall, w1, w3, w2)


def kernel(inputs, gate_w, w1, w3, w2):
    x = inputs.reshape(B, H)
    logits_t = _gate_logits(x, gate_w)
    wall = _sc_top2(logits_t.reshape(E * B)).reshape(E, B).T
    out = _experts(x, wall, w1, w3, w2)
    return out.reshape(1, 1, B, H)


# speedup vs baseline: 1.1010x; 1.0895x over previous
import functools

import jax
import jax.numpy as jnp
import numpy as np
from jax.experimental import pallas as pl
from jax.experimental.pallas import tpu as pltpu

B, H, E, F = 32, 2048, 8, 4096
BF = 512
NF = F // BF

_MASK_VAL = float(np.finfo(np.float32).min)


def _moe_kernel(x_ref, gw_ref, w1_ref, w3_ref, w2_ref, out_ref,
                wall_ref, scale_ref, xb_ref):
    e = pl.program_id(0)
    j = pl.program_id(1)

    @pl.when((e == 0) & (j == 0))
    def _gate():
        x = x_ref[...]
        xb_ref[...] = x.astype(jnp.bfloat16)
        logits = jnp.dot(x, gw_ref[...], preferred_element_type=jnp.float32)
        m0 = jnp.max(logits, axis=1, keepdims=True)
        cond0 = logits == m0
        masked = jnp.where(cond0, _MASK_VAL, logits)
        m1 = jnp.max(masked, axis=1, keepdims=True)
        cond1 = logits == m1
        pre = 1.0 / (1.0 + jnp.exp(m1 - m0))
        wall_ref[...] = (cond0.astype(jnp.float32) * pre
                         - cond1.astype(jnp.float32) * (pre - 1.0))
        out_ref[...] = jnp.zeros_like(out_ref)

    @pl.when(j == 0)
    def _scale():
        onehot = jax.lax.broadcasted_iota(jnp.int32, (1, E), 1) == e
        scale_ref[...] = jnp.sum(jnp.where(onehot, wall_ref[...], 0.0),
                                 axis=1, keepdims=True)

    xb = xb_ref[...]
    h1 = jnp.dot(xb, w1_ref[0].astype(jnp.bfloat16),
                 preferred_element_type=jnp.float32)
    h3 = jnp.dot(xb, w3_ref[0].astype(jnp.bfloat16),
                 preferred_element_type=jnp.float32)
    hidden = (h1 * jax.nn.sigmoid(h1)) * h3
    out_ref[...] += jnp.dot(hidden.astype(jnp.bfloat16),
                            w2_ref[0].astype(jnp.bfloat16),
                            preferred_element_type=jnp.float32) * scale_ref[...]


@functools.partial(jax.jit, static_argnames=("interpret",))
def _moe(x, gate_w, w1, w3, w2, interpret=False):
    return pl.pallas_call(
        _moe_kernel,
        grid=(E, NF),
        in_specs=[
            pl.BlockSpec((B, H), lambda e, j: (0, 0)),
            pl.BlockSpec((H, E), lambda e, j: (0, 0)),
            pl.BlockSpec((1, H, BF), lambda e, j: (e, 0, j)),
            pl.BlockSpec((1, H, BF), lambda e, j: (e, 0, j)),
            pl.BlockSpec((1, BF, H), lambda e, j: (e, j, 0)),
        ],
        out_specs=pl.BlockSpec((B, H), lambda e, j: (0, 0)),
        out_shape=jax.ShapeDtypeStruct((B, H), jnp.float32),
        scratch_shapes=[
            pltpu.VMEM((B, E), jnp.float32),
            pltpu.VMEM((B, 1), jnp.float32),
            pltpu.VMEM((B, H), jnp.bfloat16),
        ],
        compiler_params=pltpu.CompilerParams(
            dimension_semantics=("arbitrary", "arbitrary"),
        ),
        interpret=interpret,
    )(x, gate_w, w1, w3, w2)


def kernel(inputs, gate_w, w1, w3, w2):
    x = inputs.reshape(B, H)
    out = _moe(x, gate_w, w1, w3, w2)
    return out.reshape(1, 1, B, H)


# speedup vs baseline: 1.1037x; 1.0025x over previous
import jax
import jax.numpy as jnp
import numpy as np
from jax.experimental import pallas as pl
from jax.experimental.pallas import tpu as pltpu

B, H, E, F = 32, 2048, 8, 4096
BF = 512
NF = F // BF

_MASK_VAL = float(np.finfo(np.float32).min)


def _moe_kernel(x_ref, gw_ref, w1_ref, w3_ref, w2_ref, out_ref,
                wall_ref, scale_ref, xb_ref):
    e = pl.program_id(0)
    j = pl.program_id(1)

    @pl.when((e == 0) & (j == 0))
    def _gate():
        x = x_ref[...]
        xb_ref[...] = x.astype(jnp.bfloat16)
        logits = jnp.dot(x, gw_ref[...], preferred_element_type=jnp.float32)
        m0 = jnp.max(logits, axis=1, keepdims=True)
        cond0 = logits == m0
        masked = jnp.where(cond0, _MASK_VAL, logits)
        m1 = jnp.max(masked, axis=1, keepdims=True)
        cond1 = logits == m1
        pre = 1.0 / (1.0 + jnp.exp(m1 - m0))
        wall_ref[...] = (cond0.astype(jnp.float32) * pre
                         - cond1.astype(jnp.float32) * (pre - 1.0))
        out_ref[...] = jnp.zeros_like(out_ref)

    @pl.when(j == 0)
    def _scale():
        onehot = jax.lax.broadcasted_iota(jnp.int32, (1, E), 1) == e
        scale_ref[...] = jnp.sum(jnp.where(onehot, wall_ref[...], 0.0),
                                 axis=1, keepdims=True)

    xb = xb_ref[...]
    h1 = jnp.dot(xb, w1_ref[0].astype(jnp.bfloat16),
                 preferred_element_type=jnp.float32)
    h3 = jnp.dot(xb, w3_ref[0].astype(jnp.bfloat16),
                 preferred_element_type=jnp.float32)
    hidden = (h1 * jax.nn.sigmoid(h1)) * h3
    out_ref[...] += jnp.dot(hidden.astype(jnp.bfloat16),
                            w2_ref[0].astype(jnp.bfloat16),
                            preferred_element_type=jnp.float32) * scale_ref[...]


@jax.jit
def _moe(x, gate_w, w1, w3, w2):
    return pl.pallas_call(
        _moe_kernel,
        grid=(E, NF),
        in_specs=[
            pl.BlockSpec((B, H), lambda e, j: (0, 0)),
            pl.BlockSpec((H, E), lambda e, j: (0, 0)),
            pl.BlockSpec((1, H, BF), lambda e, j: (e, 0, j)),
            pl.BlockSpec((1, H, BF), lambda e, j: (e, 0, j)),
            pl.BlockSpec((1, BF, H), lambda e, j: (e, j, 0)),
        ],
        out_specs=pl.BlockSpec((B, H), lambda e, j: (0, 0)),
        out_shape=jax.ShapeDtypeStruct((B, H), jnp.float32),
        scratch_shapes=[
            pltpu.VMEM((B, E), jnp.float32),
            pltpu.VMEM((B, 1), jnp.float32),
            pltpu.VMEM((B, H), jnp.bfloat16),
        ],
        compiler_params=pltpu.CompilerParams(
            dimension_semantics=("arbitrary", "arbitrary"),
        ),
    )(x, gate_w, w1, w3, w2)


def kernel(inputs, gate_w, w1, w3, w2):
    x = inputs.reshape(B, H)
    out = _moe(x, gate_w, w1, w3, w2)
    return out.reshape(1, 1, B, H)
